# baseline (speedup 1.0000x reference)
import jax
import jax.numpy as jnp
from jax.experimental import pallas as pl
from jax.experimental.pallas import tpu as pltpu

T, H, I, E, K = 2048, 1024, 512, 16, 2
EB = 2
NS = E // EB
XC = 4
NC = 8


def _silu(v):
    return v * jax.nn.sigmoid(v)


def _router_weights(logits):
    cols = jax.lax.broadcasted_iota(jnp.int32, logits.shape, 1)
    m1 = jnp.max(logits, axis=-1, keepdims=True)
    i1 = jnp.min(jnp.where(logits == m1, cols, E), axis=-1, keepdims=True)
    sel1 = cols == i1
    l2 = jnp.where(sel1, -jnp.inf, logits)
    m2 = jnp.max(l2, axis=-1, keepdims=True)
    i2 = jnp.min(jnp.where(l2 == m2, cols, E), axis=-1, keepdims=True)
    sel2 = cols == i2
    e2 = jnp.exp(m2 - m1)
    return (jnp.where(sel1, 1.0, 0.0) + jnp.where(sel2, e2, 0.0)) / (1.0 + e2)


def _moe_kernel(x_hbm, bgu_hbm, bd_hbm, gate_ref, edw_ref, out_hbm,
                xs, hs, ws, acc, bgu, bd, sem_x, sem_w, sem_out):
    step = pl.program_id(0)

    def expert_rows(j, rows, nrows):
        e = step * EB + j
        r = pl.ds(rows, nrows)
        wcol = jnp.sum(
            jnp.where(jax.lax.broadcasted_iota(jnp.int32, (nrows, E), 1) == e,
                      ws[r, :], 0.0),
            axis=-1, keepdims=True)
        return jax.lax.dot_general(
            hs[r, :] * wcol, edw_ref[j], (((1,), (1,)), ((), ())),
            preferred_element_type=jnp.float32)

    @pl.when(step == 0)
    def _step0():
        XR = T // XC
        for c in range(XC):
            r = pl.ds(c * XR, XR)
            pltpu.make_async_copy(x_hbm.at[r, :], xs.at[r, :], sem_x.at[c]).start()
        for c in range(XC):
            if c == 1:
                pltpu.make_async_copy(bgu_hbm, bgu, sem_w.at[0]).start()
                pltpu.make_async_copy(bd_hbm, bd, sem_w.at[1]).start()
            r = pl.ds(c * XR, XR)
            pltpu.make_async_copy(x_hbm.at[r, :], xs.at[r, :], sem_x.at[c]).wait()
            xc = xs[r, :]
            logits = jax.lax.dot_general(
                xc, gate_ref[...], (((1,), (1,)), ((), ())),
                preferred_element_type=jnp.float32)
            ws[r, :] = _router_weights(logits)
            hs[r, :] = _silu(xc[:, :I]) * xc[:, I:]
            acc[r, :] = expert_rows(0, c * XR, XR) + expert_rows(1, c * XR, XR)

    @pl.when(step == 1)
    def _step1():
        pltpu.make_async_copy(bgu_hbm, bgu, sem_w.at[0]).wait()
        pltpu.make_async_copy(bd_hbm, bd, sem_w.at[1]).wait()
        gu = jax.lax.dot_general(
            xs[...], bgu[...], (((1,), (1,)), ((), ())),
            preferred_element_type=jnp.float32)
        act = _silu(gu[:, :I]) * gu[:, I:]
        base_y = jax.lax.dot_general(
            act, bd[...], (((1,), (1,)), ((), ())),
            preferred_element_type=jnp.float32)
        acc[...] += base_y + expert_rows(0, 0, T) + expert_rows(1, 0, T)

    @pl.when(jnp.logical_and(step > 1, step < NS - 1))
    def _step_mid():
        acc[...] += expert_rows(0, 0, T) + expert_rows(1, 0, T)

    @pl.when(step == NS - 1)
    def _step_last():
        CR = T // NC
        for c in range(NC):
            r = pl.ds(c * CR, CR)
            acc[r, :] += expert_rows(0, c * CR, CR) + expert_rows(1, c * CR, CR)
            pltpu.make_async_copy(acc.at[r, :], out_hbm.at[r, :], sem_out.at[c]).start()
        for c in range(NC):
            r = pl.ds(c * CR, CR)
            pltpu.make_async_copy(acc.at[r, :], out_hbm.at[r, :], sem_out.at[c]).wait()


@jax.jit
def kernel(x, base_gate_up_w, base_down_w, gate_w, expert_gate_up_w, expert_down_w):
    del expert_gate_up_w
    return pl.pallas_call(
        _moe_kernel,
        grid=(NS,),
        in_specs=[
            pl.BlockSpec(memory_space=pl.ANY),
            pl.BlockSpec(memory_space=pl.ANY),
            pl.BlockSpec(memory_space=pl.ANY),
            pl.BlockSpec((E, H), lambda s: (0, 0)),
            pl.BlockSpec((EB, H, I), lambda s: (s, 0, 0)),
        ],
        out_specs=pl.BlockSpec(memory_space=pl.ANY),
        out_shape=jax.ShapeDtypeStruct((T, H), jnp.float32),
        scratch_shapes=[
            pltpu.VMEM((T, H), jnp.float32),
            pltpu.VMEM((T, I), jnp.float32),
            pltpu.VMEM((T, E), jnp.float32),
            pltpu.VMEM((T, H), jnp.float32),
            pltpu.VMEM((2 * I, H), jnp.float32),
            pltpu.VMEM((H, I), jnp.float32),
            pltpu.SemaphoreType.DMA((XC,)),
            pltpu.SemaphoreType.DMA((2,)),
            pltpu.SemaphoreType.DMA((NC,)),
        ],
        compiler_params=pltpu.CompilerParams(
            dimension_semantics=("arbitrary",),
        ),
    )(x, base_gate_up_w, base_down_w, gate_w, expert_down_w)


# speedup vs baseline: 1.0128x; 1.0128x over previous
import jax
import jax.numpy as jnp
from jax.experimental import pallas as pl
from jax.experimental.pallas import tpu as pltpu

T, H, I, E, K = 2048, 1024, 512, 16, 2
EB = 2
NS = E // EB
XC = 2
NC = 8


def _silu(v):
    return v * jax.nn.sigmoid(v)


def _router_weights(logits):
    cols = jax.lax.broadcasted_iota(jnp.int32, logits.shape, 1)
    m1 = jnp.max(logits, axis=-1, keepdims=True)
    i1 = jnp.min(jnp.where(logits == m1, cols, E), axis=-1, keepdims=True)
    sel1 = cols == i1
    l2 = jnp.where(sel1, -jnp.inf, logits)
    m2 = jnp.max(l2, axis=-1, keepdims=True)
    i2 = jnp.min(jnp.where(l2 == m2, cols, E), axis=-1, keepdims=True)
    sel2 = cols == i2
    e2 = jnp.exp(m2 - m1)
    return (jnp.where(sel1, 1.0, 0.0) + jnp.where(sel2, e2, 0.0)) / (1.0 + e2)


def _moe_kernel(x_hbm, bgu_hbm, bd_hbm, gate_ref, edw_ref, out_hbm,
                xs, hs, ws, acc, bgu, bd, sem_x, sem_w, sem_out):
    step = pl.program_id(0)

    def expert_rows(j, rows, nrows):
        e = step * EB + j
        r = pl.ds(rows, nrows)
        wcol = jnp.sum(
            jnp.where(jax.lax.broadcasted_iota(jnp.int32, (nrows, E), 1) == e,
                      ws[r, :], 0.0),
            axis=-1, keepdims=True)
        return jax.lax.dot_general(
            hs[r, :] * wcol, edw_ref[j], (((1,), (1,)), ((), ())),
            preferred_element_type=jnp.float32)

    @pl.when(step == 0)
    def _step0():
        XR = T // XC
        for c in range(XC):
            r = pl.ds(c * XR, XR)
            pltpu.make_async_copy(x_hbm.at[r, :], xs.at[r, :], sem_x.at[c]).start()
        pltpu.make_async_copy(bgu_hbm, bgu, sem_w.at[0]).start()
        pltpu.make_async_copy(bd_hbm, bd, sem_w.at[1]).start()
        for c in range(XC):
            r = pl.ds(c * XR, XR)
            pltpu.make_async_copy(x_hbm.at[r, :], xs.at[r, :], sem_x.at[c]).wait()
            xc = xs[r, :]
            logits = jax.lax.dot_general(
                xc, gate_ref[...], (((1,), (1,)), ((), ())),
                preferred_element_type=jnp.float32)
            ws[r, :] = _router_weights(logits)
            hs[r, :] = _silu(xc[:, :I]) * xc[:, I:]
            acc[r, :] = expert_rows(0, c * XR, XR) + expert_rows(1, c * XR, XR)

    @pl.when(step == 1)
    def _step1():
        pltpu.make_async_copy(bgu_hbm, bgu, sem_w.at[0]).wait()
        pltpu.make_async_copy(bd_hbm, bd, sem_w.at[1]).wait()
        gu = jax.lax.dot_general(
            xs[...], bgu[...], (((1,), (1,)), ((), ())),
            preferred_element_type=jnp.float32)
        act = _silu(gu[:, :I]) * gu[:, I:]
        base_y = jax.lax.dot_general(
            act, bd[...], (((1,), (1,)), ((), ())),
            preferred_element_type=jnp.float32)
        acc[...] += base_y + expert_rows(0, 0, T) + expert_rows(1, 0, T)

    @pl.when(jnp.logical_and(step > 1, step < NS - 1))
    def _step_mid():
        acc[...] += expert_rows(0, 0, T) + expert_rows(1, 0, T)

    @pl.when(step == NS - 1)
    def _step_last():
        CR = T // NC
        for c in range(NC):
            r = pl.ds(c * CR, CR)
            acc[r, :] += expert_rows(0, c * CR, CR) + expert_rows(1, c * CR, CR)
            pltpu.make_async_copy(acc.at[r, :], out_hbm.at[r, :], sem_out.at[c]).start()
        for c in range(NC):
            r = pl.ds(c * CR, CR)
            pltpu.make_async_copy(acc.at[r, :], out_hbm.at[r, :], sem_out.at[c]).wait()


@jax.jit
def kernel(x, base_gate_up_w, base_down_w, gate_w, expert_gate_up_w, expert_down_w):
    del expert_gate_up_w
    return pl.pallas_call(
        _moe_kernel,
        grid=(NS,),
        in_specs=[
            pl.BlockSpec(memory_space=pl.ANY),
            pl.BlockSpec(memory_space=pl.ANY),
            pl.BlockSpec(memory_space=pl.ANY),
            pl.BlockSpec((E, H), lambda s: (0, 0)),
            pl.BlockSpec((EB, H, I), lambda s: (s, 0, 0)),
        ],
        out_specs=pl.BlockSpec(memory_space=pl.ANY),
        out_shape=jax.ShapeDtypeStruct((T, H), jnp.float32),
        scratch_shapes=[
            pltpu.VMEM((T, H), jnp.float32),
            pltpu.VMEM((T, I), jnp.float32),
            pltpu.VMEM((T, E), jnp.float32),
            pltpu.VMEM((T, H), jnp.float32),
            pltpu.VMEM((2 * I, H), jnp.float32),
            pltpu.VMEM((H, I), jnp.float32),
            pltpu.SemaphoreType.DMA((XC,)),
            pltpu.SemaphoreType.DMA((2,)),
            pltpu.SemaphoreType.DMA((NC,)),
        ],
        compiler_params=pltpu.CompilerParams(
            dimension_semantics=("arbitrary",),
        ),
    )(x, base_gate_up_w, base_down_w, gate_w, expert_down_w)
